# Initial kernel scaffold; baseline (speedup 1.0000x reference)
import jax
import jax.numpy as jnp
from jax.experimental import pallas as pl


def kernel(stem_w, stem_scale, stem_bias, l0_dw_w, l0_dw_scale, l0_dw_bias, l0_se_w1, l0_se_b1, l0_se_w2, l0_se_b2, l0_proj_w, l0_proj_scale, l0_proj_bias, l1_expand_w, l1_expand_scale, l1_expand_bias, l1_dw_w, l1_dw_scale, l1_dw_bias, l1_proj_w, l1_proj_scale, l1_proj_bias, l2_expand_w, l2_expand_scale, l2_expand_bias, l2_dw_w, l2_dw_scale, l2_dw_bias, l2_proj_w, l2_proj_scale, l2_proj_bias, l3_expand_w, l3_expand_scale, l3_expand_bias, l3_dw_w, l3_dw_scale, l3_dw_bias, l3_se_w1, l3_se_b1, l3_se_w2, l3_se_b2, l3_proj_w, l3_proj_scale, l3_proj_bias, last_w, last_scale, last_bias, fc1_w, fc1_b, fc2_w, fc2_b, x):
    raise NotImplementedError("write your pallas kernel here")


# speedup vs baseline: 1.1544x; 1.1544x over previous
import numpy as np

import jax
import jax.numpy as jnp
from jax.experimental import pallas as pl
from jax.experimental.pallas import tpu as pltpu

_F32 = jnp.float32
_BF16 = jnp.bfloat16


def _hardswish(y):
    return y * jnp.clip(y + 3.0, 0.0, 6.0) * (1.0 / 6.0)


def _act(y, kind):
    if kind == "relu":
        return jnp.maximum(y, 0.0)
    if kind == "hs":
        return _hardswish(y)
    return y


def _mm(x_bf, w_ref, s_ref, b_ref, kind):
    acc = jnp.dot(x_bf, w_ref[...], preferred_element_type=_F32)
    return _act(acc * s_ref[...] + b_ref[...], kind).astype(_BF16)


def _dwconv(src_ref, w_ref, s_ref, b_ref, k, stride, ho, kind):
    w = w_ref[...]
    span = (ho - 1) * stride + 1
    acc = jnp.zeros((ho, ho, 128), _F32)
    for kh in range(k):
        for kw in range(k):
            tap = src_ref[kh:kh + span:stride, kw:kw + span:stride, :]
            acc = acc + tap * w[kh * k + kw]
    y = acc * s_ref[...] + b_ref[...]
    return _act(y, kind).astype(_BF16)


def _se_gate(x_bf, hw, w1_ref, b1_ref, w2_ref, b2_ref):
    pooled = jnp.mean(x_bf.astype(_F32).reshape(hw, 128), axis=0,
                      keepdims=True)
    s1 = jnp.sum(w1_ref[...] * jnp.transpose(pooled), axis=0,
                 keepdims=True) + b1_ref[...]
    s1 = jnp.maximum(s1, 0.0)
    s2 = jnp.sum(w2_ref[...] * jnp.transpose(s1), axis=0,
                 keepdims=True) + b2_ref[...]
    return (jnp.clip(s2 + 3.0, 0.0, 6.0) * (1.0 / 6.0)).astype(_BF16)


def _store_padded(dst_ref, y_bf, pad, n):
    h = dst_ref.shape[0]
    dst_ref[0:pad, :, :] = jnp.zeros((pad, h, 128), _F32)
    dst_ref[pad + n:h, :, :] = jnp.zeros((h - pad - n, h, 128), _F32)
    dst_ref[:, 0:pad, :] = jnp.zeros((h, pad, 128), _F32)
    dst_ref[:, pad + n:h, :] = jnp.zeros((h, h - pad - n, 128), _F32)
    dst_ref[pad:pad + n, pad:pad + n, :] = y_bf.reshape(n, n, 128).astype(_F32)


def _net_kernel(*refs):
    (xs0, xs1, w4, stem_s, stem_b,
     dw0_w, dw0_s, dw0_b, se0_w1, se0_b1, se0_w2, se0_b2, p0_w, p0_s, p0_b,
     e1_w, e1_s, e1_b, dw1_w, dw1_s, dw1_b, p1_w, p1_s, p1_b,
     e2_w, e2_s, e2_b, dw2_w, dw2_s, dw2_b, p2_w, p2_s, p2_b,
     e3_w, e3_s, e3_b, dw3_w, dw3_s, dw3_b, se3_w1, se3_b1, se3_w2, se3_b2,
     p3_w, p3_s, p3_b, last_w, last_s, last_b, f1_w, f1_b, f2_w, f2_b,
     out, sc_a, sc_b, sc_c, sc_d) = refs

    acc = jnp.zeros((112 * 112, 128), _F32)
    for dh in range(2):
        for dv, xsrc in ((0, xs0), (1, xs1)):
            t = dh * 2 + dv
            tap = xsrc[0, dh * 112:dh * 112 + 112 * 112, :]
            acc = acc + jnp.dot(tap, w4[t * 16:(t + 1) * 16, :],
                                preferred_element_type=_F32)
    stem = _hardswish(acc * stem_s[...] + stem_b[...]).astype(_BF16)
    _store_padded(sc_a, stem, 1, 112)

    d0 = _dwconv(sc_a, dw0_w, dw0_s, dw0_b, 3, 2, 56, "relu")
    g0 = _se_gate(d0, 56 * 56, se0_w1, se0_b1, se0_w2, se0_b2)
    p0 = _mm((d0 * g0).reshape(56 * 56, 128), p0_w, p0_s, p0_b, "none")

    e1 = _mm(p0, e1_w, e1_s, e1_b, "relu")
    _store_padded(sc_b, e1, 1, 56)
    d1 = _dwconv(sc_b, dw1_w, dw1_s, dw1_b, 3, 2, 28, "relu")
    p1 = _mm(d1.reshape(28 * 28, 128), p1_w, p1_s, p1_b, "none")

    e2 = _mm(p1, e2_w, e2_s, e2_b, "relu")
    _store_padded(sc_c, e2, 1, 28)
    d2 = _dwconv(sc_c, dw2_w, dw2_s, dw2_b, 3, 1, 28, "relu")
    acc2 = jnp.dot(d2.reshape(28 * 28, 128), p2_w[...],
                   preferred_element_type=_F32)
    p2 = (acc2 * p2_s[...] + p2_b[...] + p1.astype(_F32)).astype(_BF16)

    e3 = _mm(p2, e3_w, e3_s, e3_b, "hs")
    _store_padded(sc_d, e3, 2, 28)
    d3 = _dwconv(sc_d, dw3_w, dw3_s, dw3_b, 5, 2, 14, "hs")
    g3 = _se_gate(d3, 14 * 14, se3_w1, se3_b1, se3_w2, se3_b2)
    p3 = _mm((d3 * g3).reshape(14 * 14, 128), p3_w, p3_s, p3_b, "none")

    lastv = _mm(p3, last_w, last_s, last_b, "hs")
    feat = jnp.mean(lastv.astype(_F32), axis=0, keepdims=True)
    h = jnp.dot(feat.astype(_BF16), f1_w[...], preferred_element_type=_F32)
    h = _hardswish(h + f1_b[...]).astype(_BF16)
    o = jnp.dot(h, f2_w[...], preferred_element_type=_F32) + f2_b[...]
    out[0] = o.astype(_BF16)


def _stem_weight_s2d(stem_w):
    idx, val = [], []
    for dh in range(2):
        for dv in range(2):
            for ph in range(2):
                for pw in range(2):
                    for ci in range(4):
                        kh, kw = 2 * dh + ph, 2 * dv + pw
                        ok = kh < 3 and kw < 3 and ci < 3
                        idx.append((kh * 3 + kw) * 3 + ci if ok else 0)
                        val.append(1.0 if ok else 0.0)
    mask = jnp.asarray(np.array(val, np.float32)[:, None]).astype(_BF16)
    return stem_w[np.array(idx)] * mask


def kernel(stem_w, stem_scale, stem_bias,
           l0_dw_w, l0_dw_scale, l0_dw_bias,
           l0_se_w1, l0_se_b1, l0_se_w2, l0_se_b2,
           l0_proj_w, l0_proj_scale, l0_proj_bias,
           l1_expand_w, l1_expand_scale, l1_expand_bias,
           l1_dw_w, l1_dw_scale, l1_dw_bias,
           l1_proj_w, l1_proj_scale, l1_proj_bias,
           l2_expand_w, l2_expand_scale, l2_expand_bias,
           l2_dw_w, l2_dw_scale, l2_dw_bias,
           l2_proj_w, l2_proj_scale, l2_proj_bias,
           l3_expand_w, l3_expand_scale, l3_expand_bias,
           l3_dw_w, l3_dw_scale, l3_dw_bias,
           l3_se_w1, l3_se_b1, l3_se_w2, l3_se_b2,
           l3_proj_w, l3_proj_scale, l3_proj_bias,
           last_w, last_scale, last_bias,
           fc1_w, fc1_b, fc2_w, fc2_b,
           x):
    n = x.shape[0]

    xt = jnp.transpose(x, (0, 2, 3, 1)).astype(_BF16)
    xp = jnp.pad(xt, ((0, 0), (1, 1), (1, 1), (0, 13)))
    xs = (xp.reshape(n, 113, 2, 113, 2, 16)
            .transpose(0, 1, 3, 2, 4, 5)[:, :, :, :, :, :4]
            .reshape(n, 113, 113, 16))
    xs0 = xs[:, :, 0:112, :].reshape(n, 113 * 112, 16)
    xs1 = xs[:, :, 1:113, :].reshape(n, 113 * 112, 16)

    w4 = _stem_weight_s2d(stem_w)

    def v(a):
        return a.reshape(1, -1)

    se0_w1 = jnp.pad(l0_se_w1, ((0, 0), (0, 120)))
    se0_b1 = v(jnp.pad(l0_se_b1, (0, 120)))
    se0_w2 = jnp.pad(l0_se_w2, ((0, 120), (0, 0)))
    se3_w1 = jnp.pad(l3_se_w1, ((0, 0), (0, 104)))
    se3_b1 = v(jnp.pad(l3_se_b1, (0, 104)))
    se3_w2 = jnp.pad(l3_se_w2, ((0, 104), (0, 0)))

    ins = [xs0, xs1, w4, v(stem_scale), v(stem_bias),
           l0_dw_w, v(l0_dw_scale), v(l0_dw_bias),
           se0_w1, se0_b1, se0_w2, v(l0_se_b2),
           l0_proj_w, v(l0_proj_scale), v(l0_proj_bias),
           l1_expand_w, v(l1_expand_scale), v(l1_expand_bias),
           l1_dw_w, v(l1_dw_scale), v(l1_dw_bias),
           l1_proj_w, v(l1_proj_scale), v(l1_proj_bias),
           l2_expand_w, v(l2_expand_scale), v(l2_expand_bias),
           l2_dw_w, v(l2_dw_scale), v(l2_dw_bias),
           l2_proj_w, v(l2_proj_scale), v(l2_proj_bias),
           l3_expand_w, v(l3_expand_scale), v(l3_expand_bias),
           l3_dw_w, v(l3_dw_scale), v(l3_dw_bias),
           se3_w1, se3_b1, se3_w2, v(l3_se_b2),
           l3_proj_w, v(l3_proj_scale), v(l3_proj_bias),
           last_w, v(last_scale), v(last_bias),
           fc1_w, v(fc1_b), fc2_w, v(fc2_b)]

    in_specs = [pl.BlockSpec((1, 113 * 112, 16), lambda i: (i, 0, 0)),
                pl.BlockSpec((1, 113 * 112, 16), lambda i: (i, 0, 0))]
    in_specs += [pl.BlockSpec(a.shape, lambda i, nd=a.ndim: (0,) * nd)
                 for a in ins[2:]]

    out = pl.pallas_call(
        _net_kernel,
        grid=(n,),
        in_specs=in_specs,
        out_specs=pl.BlockSpec((1, 1, 128), lambda i: (i, 0, 0)),
        out_shape=jax.ShapeDtypeStruct((n, 1, 128), _BF16),
        scratch_shapes=[pltpu.VMEM((114, 114, 128), _F32),
                        pltpu.VMEM((58, 58, 128), _F32),
                        pltpu.VMEM((30, 30, 128), _F32),
                        pltpu.VMEM((32, 32, 128), _F32)],
        compiler_params=pltpu.CompilerParams(
            dimension_semantics=("parallel",),
            vmem_limit_bytes=56 * 1024 * 1024),
    )(*ins)
    return out[:, 0, :10].astype(_F32)


# speedup vs baseline: 1.5063x; 1.3048x over previous
import numpy as np

import jax
import jax.numpy as jnp
from jax.experimental import pallas as pl
from jax.experimental.pallas import tpu as pltpu

_F32 = jnp.float32
_BF16 = jnp.bfloat16


def _hardswish(y):
    return y * jnp.clip(y + 3.0, 0.0, 6.0) * (1.0 / 6.0)


def _act(y, kind):
    if kind == "relu":
        return jnp.maximum(y, 0.0)
    if kind == "hs":
        return _hardswish(y)
    return y


def _mm(x_bf, w_ref, s_ref, b_ref, kind):
    acc = jnp.dot(x_bf, w_ref[...], preferred_element_type=_F32)
    return _act(acc * s_ref[...] + b_ref[...], kind).astype(_BF16)


def _dwconv(src_ref, w_ref, s_ref, b_ref, k, stride, ho, kind):
    w = w_ref[...]
    span = (ho - 1) * stride + 1
    acc = jnp.zeros((ho, ho, 128), _F32)
    for kh in range(k):
        for kw in range(k):
            tap = src_ref[kh:kh + span:stride, kw:kw + span:stride, :]
            acc = acc + tap * w[kh * k + kw]
    y = acc * s_ref[...] + b_ref[...]
    return _act(y, kind).astype(_BF16)


def _se_gate(x_bf, hw, w1_ref, b1_ref, w2_ref, b2_ref):
    pooled = jnp.mean(x_bf.astype(_F32).reshape(hw, 128), axis=0,
                      keepdims=True)
    s1 = jnp.sum(w1_ref[...] * jnp.transpose(pooled), axis=0,
                 keepdims=True) + b1_ref[...]
    s1 = jnp.maximum(s1, 0.0)
    s2 = jnp.sum(w2_ref[...] * jnp.transpose(s1), axis=0,
                 keepdims=True) + b2_ref[...]
    return (jnp.clip(s2 + 3.0, 0.0, 6.0) * (1.0 / 6.0)).astype(_BF16)


def _store_padded(dst_ref, y_bf, pad, n):
    h = dst_ref.shape[0]
    dst_ref[0:pad, :, :] = jnp.zeros((pad, h, 128), _F32)
    dst_ref[pad + n:h, :, :] = jnp.zeros((h - pad - n, h, 128), _F32)
    dst_ref[:, 0:pad, :] = jnp.zeros((h, pad, 128), _F32)
    dst_ref[:, pad + n:h, :] = jnp.zeros((h, h - pad - n, 128), _F32)
    dst_ref[pad:pad + n, pad:pad + n, :] = y_bf.reshape(n, n, 128).astype(_F32)


def _net_kernel(*refs):
    (xs0, xs1, w4, stem_s, stem_b,
     dw0_w, dw0_s, dw0_b, se0_w1, se0_b1, se0_w2, se0_b2, p0_w, p0_s, p0_b,
     e1_w, e1_s, e1_b, dw1_w, dw1_s, dw1_b, p1_w, p1_s, p1_b,
     e2_w, e2_s, e2_b, dw2_w, dw2_s, dw2_b, p2_w, p2_s, p2_b,
     e3_w, e3_s, e3_b, dw3_w, dw3_s, dw3_b, se3_w1, se3_b1, se3_w2, se3_b2,
     p3_w, p3_s, p3_b, last_w, last_s, last_b, f1_w, f1_b, f2_w, f2_b,
     out, sc_a, sc_b, sc_c, sc_d) = refs

    acc = jnp.zeros((112 * 112, 128), _F32)
    for dh in range(2):
        for dv, xsrc in ((0, xs0), (1, xs1)):
            t = dh * 2 + dv
            tap = xsrc[0, dh * 112:dh * 112 + 112 * 112, :]
            acc = acc + jnp.dot(tap, w4[t * 16:(t + 1) * 16, :],
                                preferred_element_type=_F32)
    stem = _hardswish(acc * stem_s[...] + stem_b[...]).astype(_BF16)
    _store_padded(sc_a, stem, 1, 112)

    d0 = _dwconv(sc_a, dw0_w, dw0_s, dw0_b, 3, 2, 56, "relu")
    g0 = _se_gate(d0, 56 * 56, se0_w1, se0_b1, se0_w2, se0_b2)
    p0 = _mm((d0 * g0).reshape(56 * 56, 128), p0_w, p0_s, p0_b, "none")

    e1 = _mm(p0, e1_w, e1_s, e1_b, "relu")
    _store_padded(sc_b, e1, 1, 56)
    d1 = _dwconv(sc_b, dw1_w, dw1_s, dw1_b, 3, 2, 28, "relu")
    p1 = _mm(d1.reshape(28 * 28, 128), p1_w, p1_s, p1_b, "none")

    e2 = _mm(p1, e2_w, e2_s, e2_b, "relu")
    _store_padded(sc_c, e2, 1, 28)
    d2 = _dwconv(sc_c, dw2_w, dw2_s, dw2_b, 3, 1, 28, "relu")
    acc2 = jnp.dot(d2.reshape(28 * 28, 128), p2_w[...],
                   preferred_element_type=_F32)
    p2 = (acc2 * p2_s[...] + p2_b[...] + p1.astype(_F32)).astype(_BF16)

    e3 = _mm(p2, e3_w, e3_s, e3_b, "hs")
    _store_padded(sc_d, e3, 2, 28)
    d3 = _dwconv(sc_d, dw3_w, dw3_s, dw3_b, 5, 2, 14, "hs")
    g3 = _se_gate(d3, 14 * 14, se3_w1, se3_b1, se3_w2, se3_b2)
    p3 = _mm((d3 * g3).reshape(14 * 14, 128), p3_w, p3_s, p3_b, "none")

    lastv = _mm(p3, last_w, last_s, last_b, "hs")
    feat = jnp.mean(lastv.astype(_F32), axis=0, keepdims=True)
    h = jnp.dot(feat.astype(_BF16), f1_w[...], preferred_element_type=_F32)
    h = _hardswish(h + f1_b[...]).astype(_BF16)
    o = jnp.dot(h, f2_w[...], preferred_element_type=_F32) + f2_b[...]
    out[0] = o.astype(_BF16)


def _stem_weight_s2d(stem_w):
    idx, val = [], []
    for dh in range(2):
        for dv in range(2):
            for ph in range(2):
                for pw in range(2):
                    for ci in range(3):
                        kh, kw = 2 * dh + ph, 2 * dv + pw
                        ok = kh < 3 and kw < 3
                        idx.append((kh * 3 + kw) * 3 + ci if ok else 0)
                        val.append(1.0 if ok else 0.0)
            idx += [0, 0, 0, 0]
            val += [0.0, 0.0, 0.0, 0.0]
    mask = jnp.asarray(np.array(val, np.float32)[:, None]).astype(_BF16)
    return stem_w[np.array(idx)] * mask


def kernel(stem_w, stem_scale, stem_bias,
           l0_dw_w, l0_dw_scale, l0_dw_bias,
           l0_se_w1, l0_se_b1, l0_se_w2, l0_se_b2,
           l0_proj_w, l0_proj_scale, l0_proj_bias,
           l1_expand_w, l1_expand_scale, l1_expand_bias,
           l1_dw_w, l1_dw_scale, l1_dw_bias,
           l1_proj_w, l1_proj_scale, l1_proj_bias,
           l2_expand_w, l2_expand_scale, l2_expand_bias,
           l2_dw_w, l2_dw_scale, l2_dw_bias,
           l2_proj_w, l2_proj_scale, l2_proj_bias,
           l3_expand_w, l3_expand_scale, l3_expand_bias,
           l3_dw_w, l3_dw_scale, l3_dw_bias,
           l3_se_w1, l3_se_b1, l3_se_w2, l3_se_b2,
           l3_proj_w, l3_proj_scale, l3_proj_bias,
           last_w, last_scale, last_bias,
           fc1_w, fc1_b, fc2_w, fc2_b,
           x):
    n = x.shape[0]

    xt = jnp.transpose(x, (0, 2, 3, 1)).astype(_BF16)
    xp = jnp.pad(xt, ((0, 0), (1, 1), (1, 1), (0, 0)))
    pieces0, pieces1 = [], []
    for ph in range(2):
        for pw in range(2):
            pieces0.append(xp[:, ph:ph + 225:2, pw:pw + 223:2, :])
            pieces1.append(xp[:, ph:ph + 225:2, pw + 2:pw + 225:2, :])
    z4 = jnp.zeros((n, 113, 112, 4), _BF16)
    xs0 = jnp.concatenate(pieces0 + [z4], axis=-1).reshape(n, 113 * 112, 16)
    xs1 = jnp.concatenate(pieces1 + [z4], axis=-1).reshape(n, 113 * 112, 16)

    w4 = _stem_weight_s2d(stem_w)

    def v(a):
        return a.reshape(1, -1)

    se0_w1 = jnp.pad(l0_se_w1, ((0, 0), (0, 120)))
    se0_b1 = v(jnp.pad(l0_se_b1, (0, 120)))
    se0_w2 = jnp.pad(l0_se_w2, ((0, 120), (0, 0)))
    se3_w1 = jnp.pad(l3_se_w1, ((0, 0), (0, 104)))
    se3_b1 = v(jnp.pad(l3_se_b1, (0, 104)))
    se3_w2 = jnp.pad(l3_se_w2, ((0, 104), (0, 0)))

    ins = [xs0, xs1, w4, v(stem_scale), v(stem_bias),
           l0_dw_w, v(l0_dw_scale), v(l0_dw_bias),
           se0_w1, se0_b1, se0_w2, v(l0_se_b2),
           l0_proj_w, v(l0_proj_scale), v(l0_proj_bias),
           l1_expand_w, v(l1_expand_scale), v(l1_expand_bias),
           l1_dw_w, v(l1_dw_scale), v(l1_dw_bias),
           l1_proj_w, v(l1_proj_scale), v(l1_proj_bias),
           l2_expand_w, v(l2_expand_scale), v(l2_expand_bias),
           l2_dw_w, v(l2_dw_scale), v(l2_dw_bias),
           l2_proj_w, v(l2_proj_scale), v(l2_proj_bias),
           l3_expand_w, v(l3_expand_scale), v(l3_expand_bias),
           l3_dw_w, v(l3_dw_scale), v(l3_dw_bias),
           se3_w1, se3_b1, se3_w2, v(l3_se_b2),
           l3_proj_w, v(l3_proj_scale), v(l3_proj_bias),
           last_w, v(last_scale), v(last_bias),
           fc1_w, v(fc1_b), fc2_w, v(fc2_b)]

    in_specs = [pl.BlockSpec((1, 113 * 112, 16), lambda i: (i, 0, 0)),
                pl.BlockSpec((1, 113 * 112, 16), lambda i: (i, 0, 0))]
    in_specs += [pl.BlockSpec(a.shape, lambda i, nd=a.ndim: (0,) * nd)
                 for a in ins[2:]]

    out = pl.pallas_call(
        _net_kernel,
        grid=(n,),
        in_specs=in_specs,
        out_specs=pl.BlockSpec((1, 1, 128), lambda i: (i, 0, 0)),
        out_shape=jax.ShapeDtypeStruct((n, 1, 128), _BF16),
        scratch_shapes=[pltpu.VMEM((114, 114, 128), _F32),
                        pltpu.VMEM((58, 58, 128), _F32),
                        pltpu.VMEM((30, 30, 128), _F32),
                        pltpu.VMEM((32, 32, 128), _F32)],
        compiler_params=pltpu.CompilerParams(
            dimension_semantics=("parallel",),
            vmem_limit_bytes=56 * 1024 * 1024),
    )(*ins)
    return out[:, 0, :10].astype(_F32)


# speedup vs baseline: 3.3483x; 2.2229x over previous
import numpy as np

import jax
import jax.numpy as jnp
from jax.experimental import pallas as pl
from jax.experimental.pallas import tpu as pltpu

_F32 = jnp.float32
_BF16 = jnp.bfloat16


def _hardswish(y):
    return y * jnp.clip(y + 3.0, 0.0, 6.0) * (1.0 / 6.0)


def _act(y, kind):
    if kind == "relu":
        return jnp.maximum(y, 0.0)
    if kind == "hs":
        return _hardswish(y)
    return y


def _mm(x_bf, w_ref, s_ref, b_ref, kind):
    acc = jnp.dot(x_bf, w_ref[...], preferred_element_type=_F32)
    return _act(acc * s_ref[...] + b_ref[...], kind).astype(_BF16)


def _dwconv(src_ref, w_ref, s_ref, b_ref, k, stride, ho, kind):
    w = w_ref[...]
    span = (ho - 1) * stride + 1
    acc = jnp.zeros((ho, ho, 128), _F32)
    for kh in range(k):
        for kw in range(k):
            tap = src_ref[kh:kh + span:stride, kw:kw + span:stride, :]
            acc = acc + tap * w[kh * k + kw]
    y = acc * s_ref[...] + b_ref[...]
    return _act(y, kind).astype(_BF16)


def _se_gate(x_bf, hw, w1_ref, b1_ref, w2_ref, b2_ref):
    pooled = jnp.mean(x_bf.astype(_F32).reshape(hw, 128), axis=0,
                      keepdims=True)
    s1 = jnp.sum(w1_ref[...] * jnp.transpose(pooled), axis=0,
                 keepdims=True) + b1_ref[...]
    s1 = jnp.maximum(s1, 0.0)
    s2 = jnp.sum(w2_ref[...] * jnp.transpose(s1), axis=0,
                 keepdims=True) + b2_ref[...]
    return (jnp.clip(s2 + 3.0, 0.0, 6.0) * (1.0 / 6.0)).astype(_BF16)


def _store_padded(dst_ref, y_bf, pad, n):
    h = dst_ref.shape[0]
    dst_ref[0:pad, :, :] = jnp.zeros((pad, h, 128), _F32)
    dst_ref[pad + n:h, :, :] = jnp.zeros((h - pad - n, h, 128), _F32)
    dst_ref[:, 0:pad, :] = jnp.zeros((h, pad, 128), _F32)
    dst_ref[:, pad + n:h, :] = jnp.zeros((h, h - pad - n, 128), _F32)
    dst_ref[pad:pad + n, pad:pad + n, :] = y_bf.reshape(n, n, 128).astype(_F32)


def _net_kernel(*refs):
    (xs0, xs1, w4, stem_s, stem_b,
     dw0_w, dw0_s, dw0_b, se0_w1, se0_b1, se0_w2, se0_b2, p0_w, p0_s, p0_b,
     e1_w, e1_s, e1_b, dw1_w, dw1_s, dw1_b, p1_w, p1_s, p1_b,
     e2_w, e2_s, e2_b, dw2_w, dw2_s, dw2_b, p2_w, p2_s, p2_b,
     e3_w, e3_s, e3_b, dw3_w, dw3_s, dw3_b, se3_w1, se3_b1, se3_w2, se3_b2,
     p3_w, p3_s, p3_b, last_w, last_s, last_b, f1_w, f1_b, f2_w, f2_b,
     out, sc_a, sc_b, sc_c, sc_d) = refs

    acc = jnp.zeros((112, 112, 128), _F32)
    for dh in range(2):
        for dv, xsrc in ((0, xs0), (1, xs1)):
            t = dh * 2 + dv
            lhs = xsrc[0, dh:dh + 112]
            acc = acc + jax.lax.dot_general(
                lhs, w4[t * 16:(t + 1) * 16, :],
                dimension_numbers=(((1,), (0,)), ((), ())),
                preferred_element_type=_F32)
    stem = _hardswish(acc * stem_s[...] + stem_b[...]).astype(_BF16)
    _store_padded(sc_a, stem, 1, 112)

    d0 = _dwconv(sc_a, dw0_w, dw0_s, dw0_b, 3, 2, 56, "relu")
    g0 = _se_gate(d0, 56 * 56, se0_w1, se0_b1, se0_w2, se0_b2)
    p0 = _mm((d0 * g0).reshape(56 * 56, 128), p0_w, p0_s, p0_b, "none")

    e1 = _mm(p0, e1_w, e1_s, e1_b, "relu")
    _store_padded(sc_b, e1, 1, 56)
    d1 = _dwconv(sc_b, dw1_w, dw1_s, dw1_b, 3, 2, 28, "relu")
    p1 = _mm(d1.reshape(28 * 28, 128), p1_w, p1_s, p1_b, "none")

    e2 = _mm(p1, e2_w, e2_s, e2_b, "relu")
    _store_padded(sc_c, e2, 1, 28)
    d2 = _dwconv(sc_c, dw2_w, dw2_s, dw2_b, 3, 1, 28, "relu")
    acc2 = jnp.dot(d2.reshape(28 * 28, 128), p2_w[...],
                   preferred_element_type=_F32)
    p2 = (acc2 * p2_s[...] + p2_b[...] + p1.astype(_F32)).astype(_BF16)

    e3 = _mm(p2, e3_w, e3_s, e3_b, "hs")
    _store_padded(sc_d, e3, 2, 28)
    d3 = _dwconv(sc_d, dw3_w, dw3_s, dw3_b, 5, 2, 14, "hs")
    g3 = _se_gate(d3, 14 * 14, se3_w1, se3_b1, se3_w2, se3_b2)
    p3 = _mm((d3 * g3).reshape(14 * 14, 128), p3_w, p3_s, p3_b, "none")

    lastv = _mm(p3, last_w, last_s, last_b, "hs")
    feat = jnp.mean(lastv.astype(_F32), axis=0, keepdims=True)
    h = jnp.dot(feat.astype(_BF16), f1_w[...], preferred_element_type=_F32)
    h = _hardswish(h + f1_b[...]).astype(_BF16)
    o = jnp.dot(h, f2_w[...], preferred_element_type=_F32) + f2_b[...]
    out[0] = o.astype(_BF16)


def _stem_weight_s2d(stem_w):
    idx, val = [], []
    for dh in range(2):
        for dv in range(2):
            for ph in range(2):
                for pw in range(2):
                    for ci in range(3):
                        kh, kw = 2 * dh + ph, 2 * dv + pw
                        ok = kh < 3 and kw < 3
                        idx.append((kh * 3 + kw) * 3 + ci if ok else 0)
                        val.append(1.0 if ok else 0.0)
            idx += [0, 0, 0, 0]
            val += [0.0, 0.0, 0.0, 0.0]
    mask = jnp.asarray(np.array(val, np.float32)[:, None]).astype(_BF16)
    return stem_w[np.array(idx)] * mask


def kernel(stem_w, stem_scale, stem_bias,
           l0_dw_w, l0_dw_scale, l0_dw_bias,
           l0_se_w1, l0_se_b1, l0_se_w2, l0_se_b2,
           l0_proj_w, l0_proj_scale, l0_proj_bias,
           l1_expand_w, l1_expand_scale, l1_expand_bias,
           l1_dw_w, l1_dw_scale, l1_dw_bias,
           l1_proj_w, l1_proj_scale, l1_proj_bias,
           l2_expand_w, l2_expand_scale, l2_expand_bias,
           l2_dw_w, l2_dw_scale, l2_dw_bias,
           l2_proj_w, l2_proj_scale, l2_proj_bias,
           l3_expand_w, l3_expand_scale, l3_expand_bias,
           l3_dw_w, l3_dw_scale, l3_dw_bias,
           l3_se_w1, l3_se_b1, l3_se_w2, l3_se_b2,
           l3_proj_w, l3_proj_scale, l3_proj_bias,
           last_w, last_scale, last_bias,
           fc1_w, fc1_b, fc2_w, fc2_b,
           x):
    n = x.shape[0]

    xpc = jnp.pad(x, ((0, 0), (0, 0), (1, 1), (1, 1)))
    cols0, cols1 = [], []
    for ph in range(2):
        for pw in range(2):
            for ci in range(3):
                base = xpc[:, ci]
                cols0.append(base[:, ph:ph + 225:2, pw:pw + 223:2])
                cols1.append(base[:, ph:ph + 225:2, pw + 2:pw + 225:2])
    xs0 = jnp.pad(jnp.stack(cols0, axis=2),
                  ((0, 0), (0, 0), (0, 4), (0, 0))).astype(_BF16)
    xs1 = jnp.pad(jnp.stack(cols1, axis=2),
                  ((0, 0), (0, 0), (0, 4), (0, 0))).astype(_BF16)

    w4 = _stem_weight_s2d(stem_w)

    def v(a):
        return a.reshape(1, -1)

    se0_w1 = jnp.pad(l0_se_w1, ((0, 0), (0, 120)))
    se0_b1 = v(jnp.pad(l0_se_b1, (0, 120)))
    se0_w2 = jnp.pad(l0_se_w2, ((0, 120), (0, 0)))
    se3_w1 = jnp.pad(l3_se_w1, ((0, 0), (0, 104)))
    se3_b1 = v(jnp.pad(l3_se_b1, (0, 104)))
    se3_w2 = jnp.pad(l3_se_w2, ((0, 104), (0, 0)))

    ins = [xs0, xs1, w4, v(stem_scale), v(stem_bias),
           l0_dw_w, v(l0_dw_scale), v(l0_dw_bias),
           se0_w1, se0_b1, se0_w2, v(l0_se_b2),
           l0_proj_w, v(l0_proj_scale), v(l0_proj_bias),
           l1_expand_w, v(l1_expand_scale), v(l1_expand_bias),
           l1_dw_w, v(l1_dw_scale), v(l1_dw_bias),
           l1_proj_w, v(l1_proj_scale), v(l1_proj_bias),
           l2_expand_w, v(l2_expand_scale), v(l2_expand_bias),
           l2_dw_w, v(l2_dw_scale), v(l2_dw_bias),
           l2_proj_w, v(l2_proj_scale), v(l2_proj_bias),
           l3_expand_w, v(l3_expand_scale), v(l3_expand_bias),
           l3_dw_w, v(l3_dw_scale), v(l3_dw_bias),
           se3_w1, se3_b1, se3_w2, v(l3_se_b2),
           l3_proj_w, v(l3_proj_scale), v(l3_proj_bias),
           last_w, v(last_scale), v(last_bias),
           fc1_w, v(fc1_b), fc2_w, v(fc2_b)]

    in_specs = [pl.BlockSpec((1, 113, 16, 112), lambda i: (i, 0, 0, 0)),
                pl.BlockSpec((1, 113, 16, 112), lambda i: (i, 0, 0, 0))]
    in_specs += [pl.BlockSpec(a.shape, lambda i, nd=a.ndim: (0,) * nd)
                 for a in ins[2:]]

    out = pl.pallas_call(
        _net_kernel,
        grid=(n,),
        in_specs=in_specs,
        out_specs=pl.BlockSpec((1, 1, 128), lambda i: (i, 0, 0)),
        out_shape=jax.ShapeDtypeStruct((n, 1, 128), _BF16),
        scratch_shapes=[pltpu.VMEM((114, 114, 128), _F32),
                        pltpu.VMEM((58, 58, 128), _F32),
                        pltpu.VMEM((30, 30, 128), _F32),
                        pltpu.VMEM((32, 32, 128), _F32)],
        compiler_params=pltpu.CompilerParams(
            dimension_semantics=("parallel",),
            vmem_limit_bytes=56 * 1024 * 1024),
    )(*ins)
    return out[:, 0, :10].astype(_F32)


# speedup vs baseline: 3.4056x; 1.0171x over previous
import numpy as np

import jax
import jax.numpy as jnp
from jax.experimental import pallas as pl
from jax.experimental.pallas import tpu as pltpu

_F32 = jnp.float32
_BF16 = jnp.bfloat16


def _hardswish(y):
    return y * jnp.clip(y + 3.0, 0.0, 6.0) * (1.0 / 6.0)


def _act(y, kind):
    if kind == "relu":
        return jnp.maximum(y, 0.0)
    if kind == "hs":
        return _hardswish(y)
    return y


def _mm(x_bf, w_ref, s_ref, b_ref, kind):
    acc = jnp.dot(x_bf, w_ref[...], preferred_element_type=_F32)
    return _act(acc * s_ref[...] + b_ref[...], kind).astype(_BF16)


def _dwconv(src_ref, w_ref, s_ref, b_ref, k, stride, ho, kind):
    w = w_ref[...]
    span = (ho - 1) * stride + 1
    acc = jnp.zeros((ho, ho, 128), _F32)
    for kh in range(k):
        for kw in range(k):
            tap = src_ref[kh:kh + span:stride, kw:kw + span:stride, :]
            acc = acc + tap * w[kh * k + kw]
    y = acc * s_ref[...] + b_ref[...]
    return _act(y, kind).astype(_BF16)


def _se_gate(x_bf, hw, w1_ref, b1_ref, w2_ref, b2_ref):
    pooled = jnp.mean(x_bf.astype(_F32).reshape(hw, 128), axis=0,
                      keepdims=True)
    s1 = jnp.sum(w1_ref[...] * jnp.transpose(pooled), axis=0,
                 keepdims=True) + b1_ref[...]
    s1 = jnp.maximum(s1, 0.0)
    s2 = jnp.sum(w2_ref[...] * jnp.transpose(s1), axis=0,
                 keepdims=True) + b2_ref[...]
    return (jnp.clip(s2 + 3.0, 0.0, 6.0) * (1.0 / 6.0)).astype(_BF16)


def _store_padded(dst_ref, y_bf, pad, n):
    h = dst_ref.shape[0]
    dst_ref[0:pad, :, :] = jnp.zeros((pad, h, 128), _F32)
    dst_ref[pad + n:h, :, :] = jnp.zeros((h - pad - n, h, 128), _F32)
    dst_ref[:, 0:pad, :] = jnp.zeros((h, pad, 128), _F32)
    dst_ref[:, pad + n:h, :] = jnp.zeros((h, h - pad - n, 128), _F32)
    dst_ref[pad:pad + n, pad:pad + n, :] = y_bf.reshape(n, n, 128).astype(_F32)


def _net_kernel(*refs):
    (xin, xin2, w4, stem_s, stem_b,
     dw0_w, dw0_s, dw0_b, se0_w1, se0_b1, se0_w2, se0_b2, p0_w, p0_s, p0_b,
     e1_w, e1_s, e1_b, dw1_w, dw1_s, dw1_b, p1_w, p1_s, p1_b,
     e2_w, e2_s, e2_b, dw2_w, dw2_s, dw2_b, p2_w, p2_s, p2_b,
     e3_w, e3_s, e3_b, dw3_w, dw3_s, dw3_b, se3_w1, se3_b1, se3_w2, se3_b2,
     p3_w, p3_s, p3_b, last_w, last_s, last_b, f1_w, f1_b, f2_w, f2_b,
     out, sc_a, sc_b, sc_c, sc_d) = refs

    acc = jnp.zeros((112, 112, 128), _F32)
    for dh in range(2):
        for dv, xsrc in ((0, xin), (1, xin2)):
            t = dh * 2 + dv
            lhs = xsrc[0, dh:dh + 112]
            acc = acc + jax.lax.dot_general(
                lhs, w4[t * 12:(t + 1) * 12, :],
                dimension_numbers=(((1,), (0,)), ((), ())),
                preferred_element_type=_F32)
    stem = _hardswish(acc * stem_s[...] + stem_b[...]).astype(_BF16)
    _store_padded(sc_a, stem, 1, 112)

    d0 = _dwconv(sc_a, dw0_w, dw0_s, dw0_b, 3, 2, 56, "relu")
    g0 = _se_gate(d0, 56 * 56, se0_w1, se0_b1, se0_w2, se0_b2)
    p0 = _mm((d0 * g0).reshape(56 * 56, 128), p0_w, p0_s, p0_b, "none")

    e1 = _mm(p0, e1_w, e1_s, e1_b, "relu")
    _store_padded(sc_b, e1, 1, 56)
    d1 = _dwconv(sc_b, dw1_w, dw1_s, dw1_b, 3, 2, 28, "relu")
    p1 = _mm(d1.reshape(28 * 28, 128), p1_w, p1_s, p1_b, "none")

    e2 = _mm(p1, e2_w, e2_s, e2_b, "relu")
    _store_padded(sc_c, e2, 1, 28)
    d2 = _dwconv(sc_c, dw2_w, dw2_s, dw2_b, 3, 1, 28, "relu")
    acc2 = jnp.dot(d2.reshape(28 * 28, 128), p2_w[...],
                   preferred_element_type=_F32)
    p2 = (acc2 * p2_s[...] + p2_b[...] + p1.astype(_F32)).astype(_BF16)

    e3 = _mm(p2, e3_w, e3_s, e3_b, "hs")
    _store_padded(sc_d, e3, 2, 28)
    d3 = _dwconv(sc_d, dw3_w, dw3_s, dw3_b, 5, 2, 14, "hs")
    g3 = _se_gate(d3, 14 * 14, se3_w1, se3_b1, se3_w2, se3_b2)
    p3 = _mm((d3 * g3).reshape(14 * 14, 128), p3_w, p3_s, p3_b, "none")

    lastv = _mm(p3, last_w, last_s, last_b, "hs")
    feat = jnp.mean(lastv.astype(_F32), axis=0, keepdims=True)
    h = jnp.dot(feat.astype(_BF16), f1_w[...], preferred_element_type=_F32)
    h = _hardswish(h + f1_b[...]).astype(_BF16)
    o = jnp.dot(h, f2_w[...], preferred_element_type=_F32) + f2_b[...]
    out[0] = o.astype(_BF16)


def _stem_weight_s2d(stem_w):
    idx, val = [], []
    for dh in range(2):
        for dv in range(2):
            for ph in range(2):
                for pw in range(2):
                    for ci in range(3):
                        kh, kw = 2 * dh + ph, 2 * dv + pw
                        ok = kh < 3 and kw < 3
                        idx.append((kh * 3 + kw) * 3 + ci if ok else 0)
                        val.append(1.0 if ok else 0.0)
    mask = jnp.asarray(np.array(val, np.float32)[:, None]).astype(_BF16)
    return stem_w[np.array(idx)] * mask


def kernel(stem_w, stem_scale, stem_bias,
           l0_dw_w, l0_dw_scale, l0_dw_bias,
           l0_se_w1, l0_se_b1, l0_se_w2, l0_se_b2,
           l0_proj_w, l0_proj_scale, l0_proj_bias,
           l1_expand_w, l1_expand_scale, l1_expand_bias,
           l1_dw_w, l1_dw_scale, l1_dw_bias,
           l1_proj_w, l1_proj_scale, l1_proj_bias,
           l2_expand_w, l2_expand_scale, l2_expand_bias,
           l2_dw_w, l2_dw_scale, l2_dw_bias,
           l2_proj_w, l2_proj_scale, l2_proj_bias,
           l3_expand_w, l3_expand_scale, l3_expand_bias,
           l3_dw_w, l3_dw_scale, l3_dw_bias,
           l3_se_w1, l3_se_b1, l3_se_w2, l3_se_b2,
           l3_proj_w, l3_proj_scale, l3_proj_bias,
           last_w, last_scale, last_bias,
           fc1_w, fc1_b, fc2_w, fc2_b,
           x):
    n = x.shape[0]

    xpc = jnp.pad(x.astype(_BF16),
                  ((0, 0), (0, 0), (1, 1), (1, 1)))
    cols0, cols1 = [], []
    for ph in range(2):
        for pw in range(2):
            for ci in range(3):
                base = xpc[:, ci]
                cols0.append(base[:, ph:ph + 225:2, pw:pw + 223:2])
                cols1.append(base[:, ph:ph + 225:2, pw + 2:pw + 225:2])
    xs0 = jnp.stack(cols0, axis=2)
    xs1 = jnp.stack(cols1, axis=2)

    w4 = _stem_weight_s2d(stem_w)

    def v(a):
        return a.reshape(1, -1)

    se0_w1 = jnp.pad(l0_se_w1, ((0, 0), (0, 120)))
    se0_b1 = v(jnp.pad(l0_se_b1, (0, 120)))
    se0_w2 = jnp.pad(l0_se_w2, ((0, 120), (0, 0)))
    se3_w1 = jnp.pad(l3_se_w1, ((0, 0), (0, 104)))
    se3_b1 = v(jnp.pad(l3_se_b1, (0, 104)))
    se3_w2 = jnp.pad(l3_se_w2, ((0, 104), (0, 0)))

    ins = [xs0, xs1, w4, v(stem_scale), v(stem_bias),
           l0_dw_w, v(l0_dw_scale), v(l0_dw_bias),
           se0_w1, se0_b1, se0_w2, v(l0_se_b2),
           l0_proj_w, v(l0_proj_scale), v(l0_proj_bias),
           l1_expand_w, v(l1_expand_scale), v(l1_expand_bias),
           l1_dw_w, v(l1_dw_scale), v(l1_dw_bias),
           l1_proj_w, v(l1_proj_scale), v(l1_proj_bias),
           l2_expand_w, v(l2_expand_scale), v(l2_expand_bias),
           l2_dw_w, v(l2_dw_scale), v(l2_dw_bias),
           l2_proj_w, v(l2_proj_scale), v(l2_proj_bias),
           l3_expand_w, v(l3_expand_scale), v(l3_expand_bias),
           l3_dw_w, v(l3_dw_scale), v(l3_dw_bias),
           se3_w1, se3_b1, se3_w2, v(l3_se_b2),
           l3_proj_w, v(l3_proj_scale), v(l3_proj_bias),
           last_w, v(last_scale), v(last_bias),
           fc1_w, v(fc1_b), fc2_w, v(fc2_b)]

    in_specs = [pl.BlockSpec((1, 113, 12, 112), lambda i: (i, 0, 0, 0)),
                pl.BlockSpec((1, 113, 12, 112), lambda i: (i, 0, 0, 0))]
    in_specs += [pl.BlockSpec(a.shape, lambda i, nd=a.ndim: (0,) * nd)
                 for a in ins[2:]]

    out = pl.pallas_call(
        _net_kernel,
        grid=(n,),
        in_specs=in_specs,
        out_specs=pl.BlockSpec((1, 1, 128), lambda i: (i, 0, 0)),
        out_shape=jax.ShapeDtypeStruct((n, 1, 128), _BF16),
        scratch_shapes=[pltpu.VMEM((114, 114, 128), _F32),
                        pltpu.VMEM((58, 58, 128), _F32),
                        pltpu.VMEM((30, 30, 128), _F32),
                        pltpu.VMEM((32, 32, 128), _F32)],
        compiler_params=pltpu.CompilerParams(
            dimension_semantics=("parallel",),
            vmem_limit_bytes=56 * 1024 * 1024),
    )(*ins)
    return out[:, 0, :10].astype(_F32)


# speedup vs baseline: 12.2136x; 3.5864x over previous
import numpy as np

import jax
import jax.numpy as jnp
from jax.experimental import pallas as pl
from jax.experimental.pallas import tpu as pltpu

_F32 = jnp.float32
_BF16 = jnp.bfloat16


def _hardswish(y):
    return y * jnp.clip(y + 3.0, 0.0, 6.0) * (1.0 / 6.0)


def _act(y, kind):
    if kind == "relu":
        return jnp.maximum(y, 0.0)
    if kind == "hs":
        return _hardswish(y)
    return y


def _mm(x_bf, w_ref, s_ref, b_ref, kind):
    acc = jnp.dot(x_bf, w_ref[...], preferred_element_type=_F32)
    return _act(acc * s_ref[...] + b_ref[...], kind).astype(_BF16)


def _dwconv(src_ref, w_ref, s_ref, b_ref, k, stride, ho, kind):
    w = w_ref[...]
    span = (ho - 1) * stride + 1
    acc = jnp.zeros((ho, ho, 128), _F32)
    for kh in range(k):
        for kw in range(k):
            tap = src_ref[kh:kh + span:stride, kw:kw + span:stride, :]
            acc = acc + tap * w[kh * k + kw]
    y = acc * s_ref[...] + b_ref[...]
    return _act(y, kind).astype(_BF16)


def _se_gate(x_bf, hw, w1_ref, b1_ref, w2_ref, b2_ref):
    pooled = jnp.mean(x_bf.astype(_F32).reshape(hw, 128), axis=0,
                      keepdims=True)
    s1 = jnp.sum(w1_ref[...] * jnp.transpose(pooled), axis=0,
                 keepdims=True) + b1_ref[...]
    s1 = jnp.maximum(s1, 0.0)
    s2 = jnp.sum(w2_ref[...] * jnp.transpose(s1), axis=0,
                 keepdims=True) + b2_ref[...]
    return (jnp.clip(s2 + 3.0, 0.0, 6.0) * (1.0 / 6.0)).astype(_BF16)


def _store_padded(dst_ref, y_bf, pad, n):
    h = dst_ref.shape[0]
    dst_ref[0:pad, :, :] = jnp.zeros((pad, h, 128), _F32)
    dst_ref[pad + n:h, :, :] = jnp.zeros((h - pad - n, h, 128), _F32)
    dst_ref[:, 0:pad, :] = jnp.zeros((h, pad, 128), _F32)
    dst_ref[:, pad + n:h, :] = jnp.zeros((h, h - pad - n, 128), _F32)
    dst_ref[pad:pad + n, pad:pad + n, :] = y_bf.reshape(n, n, 128).astype(_F32)


def _net_kernel(*refs):
    (xin, esel, w4, stem_s, stem_b,
     dw0_w, dw0_s, dw0_b, se0_w1, se0_b1, se0_w2, se0_b2, p0_w, p0_s, p0_b,
     e1_w, e1_s, e1_b, dw1_w, dw1_s, dw1_b, p1_w, p1_s, p1_b,
     e2_w, e2_s, e2_b, dw2_w, dw2_s, dw2_b, p2_w, p2_s, p2_b,
     e3_w, e3_s, e3_b, dw3_w, dw3_s, dw3_b, se3_w1, se3_b1, se3_w2, se3_b2,
     p3_w, p3_s, p3_b, last_w, last_s, last_b, f1_w, f1_b, f2_w, f2_b,
     out, sc_a, sc_b, sc_c, sc_d, se_sc, sp0, sp1) = refs

    for ci in range(3):
        xc = xin[0, ci].astype(_BF16)
        for q in range(2):
            se_sc[ci, q, :, 0:112] = jnp.dot(
                xc, esel[q], preferred_element_type=_F32)
    for dv, sp in ((0, sp0), (1, sp1)):
        sp[...] = jnp.zeros((16, 113, 128), _BF16)
        k = 0
        for ph in range(2):
            rs, r0 = ((slice(1, 224, 2), 1) if ph == 0
                      else (slice(0, 223, 2), 0))
            for pw in range(2):
                q = 1 - pw
                if pw == 0 and dv == 0:
                    c0, ncol, u0 = 1, 111, 0
                elif pw == 1 and dv == 1:
                    c0, ncol, u0 = 0, 111, 1
                else:
                    c0, ncol, u0 = 0, 112, 0
                for ci in range(3):
                    rows = se_sc[ci, q, rs, 0:112]
                    sp[k, r0:r0 + 112, c0:c0 + ncol] = (
                        rows[:, u0:u0 + ncol].astype(_BF16))
                    k += 1

    acc = jnp.zeros((112, 112, 128), _F32)
    for dh in range(2):
        for dv, sp in ((0, sp0), (1, sp1)):
            t = dh * 2 + dv
            full = jax.lax.dot_general(
                sp[...], w4[t * 16:(t + 1) * 16, :],
                dimension_numbers=(((0,), (0,)), ((), ())),
                preferred_element_type=_F32)
            acc = acc + full[dh:dh + 112, 0:112, :]
    stem = _hardswish(acc * stem_s[...] + stem_b[...]).astype(_BF16)
    _store_padded(sc_a, stem, 1, 112)

    d0 = _dwconv(sc_a, dw0_w, dw0_s, dw0_b, 3, 2, 56, "relu")
    g0 = _se_gate(d0, 56 * 56, se0_w1, se0_b1, se0_w2, se0_b2)
    p0 = _mm((d0 * g0).reshape(56 * 56, 128), p0_w, p0_s, p0_b, "none")

    e1 = _mm(p0, e1_w, e1_s, e1_b, "relu")
    _store_padded(sc_b, e1, 1, 56)
    d1 = _dwconv(sc_b, dw1_w, dw1_s, dw1_b, 3, 2, 28, "relu")
    p1 = _mm(d1.reshape(28 * 28, 128), p1_w, p1_s, p1_b, "none")

    e2 = _mm(p1, e2_w, e2_s, e2_b, "relu")
    _store_padded(sc_c, e2, 1, 28)
    d2 = _dwconv(sc_c, dw2_w, dw2_s, dw2_b, 3, 1, 28, "relu")
    acc2 = jnp.dot(d2.reshape(28 * 28, 128), p2_w[...],
                   preferred_element_type=_F32)
    p2 = (acc2 * p2_s[...] + p2_b[...] + p1.astype(_F32)).astype(_BF16)

    e3 = _mm(p2, e3_w, e3_s, e3_b, "hs")
    _store_padded(sc_d, e3, 2, 28)
    d3 = _dwconv(sc_d, dw3_w, dw3_s, dw3_b, 5, 2, 14, "hs")
    g3 = _se_gate(d3, 14 * 14, se3_w1, se3_b1, se3_w2, se3_b2)
    p3 = _mm((d3 * g3).reshape(14 * 14, 128), p3_w, p3_s, p3_b, "none")

    lastv = _mm(p3, last_w, last_s, last_b, "hs")
    feat = jnp.mean(lastv.astype(_F32), axis=0, keepdims=True)
    h = jnp.dot(feat.astype(_BF16), f1_w[...], preferred_element_type=_F32)
    h = _hardswish(h + f1_b[...]).astype(_BF16)
    o = jnp.dot(h, f2_w[...], preferred_element_type=_F32) + f2_b[...]
    out[0] = o.astype(_BF16)


def _stem_weight_s2d(stem_w):
    idx, val = [], []
    for dh in range(2):
        for dv in range(2):
            for ph in range(2):
                for pw in range(2):
                    for ci in range(3):
                        kh, kw = 2 * dh + ph, 2 * dv + pw
                        ok = kh < 3 and kw < 3
                        idx.append((kh * 3 + kw) * 3 + ci if ok else 0)
                        val.append(1.0 if ok else 0.0)
            idx += [0, 0, 0, 0]
            val += [0.0, 0.0, 0.0, 0.0]
    mask = jnp.asarray(np.array(val, np.float32)[:, None]).astype(_BF16)
    return stem_w[np.array(idx)] * mask


def kernel(stem_w, stem_scale, stem_bias,
           l0_dw_w, l0_dw_scale, l0_dw_bias,
           l0_se_w1, l0_se_b1, l0_se_w2, l0_se_b2,
           l0_proj_w, l0_proj_scale, l0_proj_bias,
           l1_expand_w, l1_expand_scale, l1_expand_bias,
           l1_dw_w, l1_dw_scale, l1_dw_bias,
           l1_proj_w, l1_proj_scale, l1_proj_bias,
           l2_expand_w, l2_expand_scale, l2_expand_bias,
           l2_dw_w, l2_dw_scale, l2_dw_bias,
           l2_proj_w, l2_proj_scale, l2_proj_bias,
           l3_expand_w, l3_expand_scale, l3_expand_bias,
           l3_dw_w, l3_dw_scale, l3_dw_bias,
           l3_se_w1, l3_se_b1, l3_se_w2, l3_se_b2,
           l3_proj_w, l3_proj_scale, l3_proj_bias,
           last_w, last_scale, last_bias,
           fc1_w, fc1_b, fc2_w, fc2_b,
           x):
    n = x.shape[0]

    esel_np = np.zeros((2, 224, 112), np.float32)
    for q in range(2):
        for u in range(112):
            esel_np[q, 2 * u + q, u] = 1.0
    esel = jnp.asarray(esel_np).astype(_BF16)

    w4 = _stem_weight_s2d(stem_w)

    def v(a):
        return a.reshape(1, -1)

    se0_w1 = jnp.pad(l0_se_w1, ((0, 0), (0, 120)))
    se0_b1 = v(jnp.pad(l0_se_b1, (0, 120)))
    se0_w2 = jnp.pad(l0_se_w2, ((0, 120), (0, 0)))
    se3_w1 = jnp.pad(l3_se_w1, ((0, 0), (0, 104)))
    se3_b1 = v(jnp.pad(l3_se_b1, (0, 104)))
    se3_w2 = jnp.pad(l3_se_w2, ((0, 104), (0, 0)))

    ins = [x, esel, w4, v(stem_scale), v(stem_bias),
           l0_dw_w, v(l0_dw_scale), v(l0_dw_bias),
           se0_w1, se0_b1, se0_w2, v(l0_se_b2),
           l0_proj_w, v(l0_proj_scale), v(l0_proj_bias),
           l1_expand_w, v(l1_expand_scale), v(l1_expand_bias),
           l1_dw_w, v(l1_dw_scale), v(l1_dw_bias),
           l1_proj_w, v(l1_proj_scale), v(l1_proj_bias),
           l2_expand_w, v(l2_expand_scale), v(l2_expand_bias),
           l2_dw_w, v(l2_dw_scale), v(l2_dw_bias),
           l2_proj_w, v(l2_proj_scale), v(l2_proj_bias),
           l3_expand_w, v(l3_expand_scale), v(l3_expand_bias),
           l3_dw_w, v(l3_dw_scale), v(l3_dw_bias),
           se3_w1, se3_b1, se3_w2, v(l3_se_b2),
           l3_proj_w, v(l3_proj_scale), v(l3_proj_bias),
           last_w, v(last_scale), v(last_bias),
           fc1_w, v(fc1_b), fc2_w, v(fc2_b)]

    in_specs = [pl.BlockSpec((1, 3, 224, 224), lambda i: (i, 0, 0, 0))]
    in_specs += [pl.BlockSpec(a.shape, lambda i, nd=a.ndim: (0,) * nd)
                 for a in ins[1:]]

    out = pl.pallas_call(
        _net_kernel,
        grid=(n,),
        in_specs=in_specs,
        out_specs=pl.BlockSpec((1, 1, 128), lambda i: (i, 0, 0)),
        out_shape=jax.ShapeDtypeStruct((n, 1, 128), _BF16),
        scratch_shapes=[pltpu.VMEM((114, 114, 128), _F32),
                        pltpu.VMEM((58, 58, 128), _F32),
                        pltpu.VMEM((30, 30, 128), _F32),
                        pltpu.VMEM((32, 32, 128), _F32),
                        pltpu.VMEM((3, 2, 224, 128), _F32),
                        pltpu.VMEM((16, 113, 128), _BF16),
                        pltpu.VMEM((16, 113, 128), _BF16)],
        compiler_params=pltpu.CompilerParams(
            dimension_semantics=("parallel",),
            vmem_limit_bytes=56 * 1024 * 1024),
    )(*ins)
    return out[:, 0, :10].astype(_F32)
